# Initial kernel scaffold; baseline (speedup 1.0000x reference)
import jax
import jax.numpy as jnp
from jax.experimental import pallas as pl


def kernel(prefix, embedding_weight):
    raise NotImplementedError("write your pallas kernel here")


# speedup vs baseline: 1.0801x; 1.0801x over previous
import functools

import jax
import jax.numpy as jnp
from jax import lax
from jax.experimental import pallas as pl
from jax.experimental.pallas import tpu as pltpu
from jax.experimental.pallas import tpu_sc as plsc

NUM_VIRTUAL_TOKENS = 20
TOKEN_DIM = 768
NUM_LAYERS = 12
EMBED_DIM = 2 * NUM_LAYERS * TOKEN_DIM
BATCH = 128

_INFO = plsc.get_sparse_core_info()
_NC = _INFO.num_cores
_NS = _INFO.num_subcores
_NW = _NC * _NS

_B = BATCH * NUM_VIRTUAL_TOKENS
_B_PER_W = _B // _NW
_G = 4
_STEPS = _B_PER_W // _G


@functools.partial(
    pl.kernel,
    mesh=plsc.VectorSubcoreMesh(core_axis_name="c", subcore_axis_name="s"),
    out_type=jax.ShapeDtypeStruct((_B, EMBED_DIM), jnp.float32),
    scratch_types=[
        pltpu.VMEM((_STEPS, _G), jnp.int32),
        pltpu.VMEM((_G, EMBED_DIM), jnp.float32),
        pltpu.SemaphoreType.DMA,
    ],
)
def _sc_gather(idx_hbm, table_hbm, out_hbm, idx_v, rows_v, sem):
    wid = lax.axis_index("s") * _NC + lax.axis_index("c")
    base = wid * _B_PER_W
    pltpu.sync_copy(idx_hbm.at[wid], idx_v)

    def step(j, carry):
        pltpu.async_copy(table_hbm.at[idx_v.at[j]], rows_v, sem).wait()
        pltpu.sync_copy(rows_v, out_hbm.at[pl.ds(base + j * _G, _G)])
        return carry

    lax.fori_loop(0, _STEPS, step, 0)


def kernel(prefix, embedding_weight):
    idx = prefix.astype(jnp.int32).reshape(_NW, _STEPS, _G)
    out = _sc_gather(idx, embedding_weight)
    return out.reshape(BATCH, NUM_VIRTUAL_TOKENS, EMBED_DIM)


# speedup vs baseline: 1.0932x; 1.0121x over previous
import functools

import jax
import jax.numpy as jnp
from jax import lax
from jax.experimental import pallas as pl
from jax.experimental.pallas import tpu as pltpu
from jax.experimental.pallas import tpu_sc as plsc

NUM_VIRTUAL_TOKENS = 20
TOKEN_DIM = 768
NUM_LAYERS = 12
EMBED_DIM = 2 * NUM_LAYERS * TOKEN_DIM
BATCH = 128

_INFO = plsc.get_sparse_core_info()
_NC = _INFO.num_cores
_NS = _INFO.num_subcores
_NW = _NC * _NS

_B = BATCH * NUM_VIRTUAL_TOKENS
_B_PER_W = _B // _NW
_G = 2
_STEPS = _B_PER_W // _G
_NBUF = 2
_K = _STEPS // _NBUF


@functools.partial(
    pl.kernel,
    mesh=plsc.VectorSubcoreMesh(core_axis_name="c", subcore_axis_name="s"),
    out_type=jax.ShapeDtypeStruct((_B, EMBED_DIM), jnp.float32),
    scratch_types=[
        pltpu.VMEM((_STEPS, _G), jnp.int32),
        pltpu.VMEM((_NBUF, _G, EMBED_DIM), jnp.float32),
        pltpu.SemaphoreType.DMA,
        pltpu.SemaphoreType.DMA,
    ],
)
def _sc_gather(idx_hbm, table_hbm, out_hbm, idx_v, rows_v, gsem, ssem):
    wid = lax.axis_index("s") * _NC + lax.axis_index("c")
    base = wid * _B_PER_W
    pltpu.sync_copy(idx_hbm.at[wid], idx_v)

    def gather_start(jb, b):
        pltpu.async_copy(table_hbm.at[idx_v.at[jb]], rows_v.at[b], gsem)

    def gather_wait(jb, b):
        pltpu.make_async_copy(table_hbm.at[idx_v.at[jb]], rows_v.at[b], gsem).wait()

    def scatter_start(jb, b):
        pltpu.async_copy(rows_v.at[b], out_hbm.at[pl.ds(base + jb * _G, _G)], ssem)

    def scatter_wait(jb, b):
        pltpu.make_async_copy(rows_v.at[b], out_hbm.at[pl.ds(base + jb * _G, _G)], ssem).wait()

    for b in range(_NBUF):
        gather_start(b, b)

    def body(k, carry):
        for b in range(_NBUF):
            jb = k * _NBUF + b
            gather_wait(jb, b)
            scatter_start(jb, b)
        for b in range(_NBUF):
            jb = k * _NBUF + b
            scatter_wait(jb, b)
            gather_start(jb + _NBUF, b)
        return carry

    lax.fori_loop(0, _K - 1, body, 0)

    for b in range(_NBUF):
        jb = (_K - 1) * _NBUF + b
        gather_wait(jb, b)
        scatter_start(jb, b)
    for b in range(_NBUF):
        jb = (_K - 1) * _NBUF + b
        scatter_wait(jb, b)


def kernel(prefix, embedding_weight):
    idx = prefix.astype(jnp.int32).reshape(_NW, _STEPS, _G)
    out = _sc_gather(idx, embedding_weight)
    return out.reshape(BATCH, NUM_VIRTUAL_TOKENS, EMBED_DIM)


# speedup vs baseline: 1.5006x; 1.3727x over previous
import functools

import jax
import jax.numpy as jnp
from jax import lax
from jax.experimental import pallas as pl
from jax.experimental.pallas import tpu as pltpu
from jax.experimental.pallas import tpu_sc as plsc

NUM_VIRTUAL_TOKENS = 20
TOKEN_DIM = 768
NUM_LAYERS = 12
EMBED_DIM = 2 * NUM_LAYERS * TOKEN_DIM
BATCH = 128

_INFO = plsc.get_sparse_core_info()
_NC = _INFO.num_cores
_NS = _INFO.num_subcores
_NW = _NC * _NS

_B = BATCH * NUM_VIRTUAL_TOKENS
_B_PER_W = _B // _NW
_G = 2
_STEPS = _B_PER_W // _G
_NBUF = 2
_K = _STEPS // _NBUF


@functools.partial(
    pl.kernel,
    mesh=plsc.VectorSubcoreMesh(core_axis_name="c", subcore_axis_name="s"),
    out_type=jax.ShapeDtypeStruct((BATCH, NUM_VIRTUAL_TOKENS, EMBED_DIM), jnp.float32),
    scratch_types=[
        pltpu.VMEM((_STEPS, _G), jnp.int32),
        pltpu.VMEM((_NBUF, _G, EMBED_DIM), jnp.float32),
        pltpu.SemaphoreType.DMA,
        pltpu.SemaphoreType.DMA,
    ],
)
def _sc_gather(idx_hbm, table_hbm, out_hbm, idx_v, rows_v, gsem, ssem):
    wid = lax.axis_index("s") * _NC + lax.axis_index("c")
    pltpu.sync_copy(idx_hbm.at[wid], idx_v)

    def _dst(jb):
        return out_hbm.at[4 * wid + jb // 10, pl.ds((jb % 10) * _G, _G)]

    def gather_start(jb, b):
        pltpu.async_copy(table_hbm.at[idx_v.at[jb]], rows_v.at[b], gsem)

    def gather_wait(jb, b):
        pltpu.make_async_copy(table_hbm.at[idx_v.at[jb]], rows_v.at[b], gsem).wait()

    def scatter_start(jb, b):
        pltpu.async_copy(rows_v.at[b], _dst(jb), ssem)

    def scatter_wait(jb, b):
        pltpu.make_async_copy(rows_v.at[b], _dst(jb), ssem).wait()

    for b in range(_NBUF):
        gather_start(b, b)

    def body(k, carry):
        for b in range(_NBUF):
            jb = k * _NBUF + b
            gather_wait(jb, b)
            scatter_start(jb, b)
        for b in range(_NBUF):
            jb = k * _NBUF + b
            scatter_wait(jb, b)
            gather_start(jb + _NBUF, b)
        return carry

    lax.fori_loop(0, _K - 1, body, 0)

    for b in range(_NBUF):
        jb = (_K - 1) * _NBUF + b
        gather_wait(jb, b)
        scatter_start(jb, b)
    for b in range(_NBUF):
        jb = (_K - 1) * _NBUF + b
        scatter_wait(jb, b)


def kernel(prefix, embedding_weight):
    idx = prefix.astype(jnp.int32).reshape(_NW, _STEPS, _G)
    return _sc_gather(idx, embedding_weight)


# speedup vs baseline: 2.7369x; 1.8238x over previous
import functools

import jax
import jax.numpy as jnp
from jax import lax
from jax.experimental import pallas as pl
from jax.experimental.pallas import tpu as pltpu
from jax.experimental.pallas import tpu_sc as plsc

NUM_VIRTUAL_TOKENS = 20
TOKEN_DIM = 768
NUM_LAYERS = 12
EMBED_DIM = 2 * NUM_LAYERS * TOKEN_DIM
BATCH = 128

_INFO = plsc.get_sparse_core_info()
_NC = _INFO.num_cores
_NS = _INFO.num_subcores
_NW = _NC * _NS

_B = BATCH * NUM_VIRTUAL_TOKENS
_B_PER_W = _B // _NW
_G = 2
_STEPS = _B_PER_W // _G
_NBUF = 2
_K = _STEPS // _NBUF


@functools.partial(
    pl.kernel,
    mesh=plsc.VectorSubcoreMesh(core_axis_name="c", subcore_axis_name="s"),
    out_type=jax.ShapeDtypeStruct((_B, EMBED_DIM), jnp.float32),
    scratch_types=[
        pltpu.VMEM((_STEPS, _G), jnp.int32),
        pltpu.VMEM((_NBUF, _G, EMBED_DIM), jnp.float32),
        pltpu.SemaphoreType.DMA,
        pltpu.SemaphoreType.DMA,
    ],
)
def _sc_gather(idx_hbm, table_hbm, out_hbm, idx_v, rows_v, gsem, ssem):
    wid = lax.axis_index("s") * _NC + lax.axis_index("c")
    base = wid * _B_PER_W
    pltpu.sync_copy(idx_hbm.at[wid], idx_v)

    def _dst(jb):
        return out_hbm.at[pl.ds(base + jb * _G, _G)]

    def gather_start(jb, b):
        pltpu.async_copy(table_hbm.at[idx_v.at[jb]], rows_v.at[b], gsem)

    def gather_wait(jb, b):
        pltpu.make_async_copy(table_hbm.at[idx_v.at[jb]], rows_v.at[b], gsem).wait()

    def scatter_start(jb, b):
        pltpu.async_copy(rows_v.at[b], _dst(jb), ssem)

    def scatter_wait(jb, b):
        pltpu.make_async_copy(rows_v.at[b], _dst(jb), ssem).wait()

    for b in range(_NBUF):
        gather_start(b, b)

    def body(k, carry):
        for b in range(_NBUF):
            jb = k * _NBUF + b
            gather_wait(jb, b)
            scatter_start(jb, b)
        for b in range(_NBUF):
            jb = k * _NBUF + b
            scatter_wait(jb, b)
            gather_start(jb + _NBUF, b)
        return carry

    lax.fori_loop(0, _K - 1, body, 0)

    for b in range(_NBUF):
        jb = (_K - 1) * _NBUF + b
        gather_wait(jb, b)
        scatter_start(jb, b)
    for b in range(_NBUF):
        jb = (_K - 1) * _NBUF + b
        scatter_wait(jb, b)


def kernel(prefix, embedding_weight):
    idx = prefix.astype(jnp.int32).T.reshape(_NW, _STEPS, _G)
    out = _sc_gather(idx, embedding_weight)
    out = out.reshape(NUM_VIRTUAL_TOKENS, BATCH, EMBED_DIM)
    return out.transpose(1, 0, 2)


# speedup vs baseline: 5.5079x; 2.0125x over previous
import functools

import jax
import jax.numpy as jnp
from jax import lax
from jax.experimental import pallas as pl
from jax.experimental.pallas import tpu as pltpu
from jax.experimental.pallas import tpu_sc as plsc

NUM_VIRTUAL_TOKENS = 20
TOKEN_DIM = 768
NUM_LAYERS = 12
EMBED_DIM = 2 * NUM_LAYERS * TOKEN_DIM
BATCH = 128

_INFO = plsc.get_sparse_core_info()
_NC = _INFO.num_cores
_NS = _INFO.num_subcores
_NW = _NC * _NS

_B = BATCH * NUM_VIRTUAL_TOKENS
_NG = 8
_NCH = _NW // _NG
_ROWS_PER_G = _B // _NG
_DC = EMBED_DIM // _NCH


@functools.partial(
    pl.kernel,
    mesh=plsc.VectorSubcoreMesh(core_axis_name="c", subcore_axis_name="s"),
    out_type=jax.ShapeDtypeStruct((_B, EMBED_DIM), jnp.float32),
    scratch_types=[
        pltpu.VMEM((_ROWS_PER_G,), jnp.int32),
        pltpu.VMEM((NUM_VIRTUAL_TOKENS, _DC), jnp.float32),
        pltpu.SemaphoreType.DMA,
    ],
)
def _sc_gather(idx_hbm, table_hbm, out_hbm, idx_v, tab_v, sem):
    wid = lax.axis_index("s") * _NC + lax.axis_index("c")
    g = wid // _NCH
    c = wid % _NCH
    col0 = c * _DC
    base = g * _ROWS_PER_G

    pltpu.sync_copy(idx_hbm.at[g], idx_v)
    pltpu.sync_copy(table_hbm.at[:, pl.ds(col0, _DC)], tab_v)

    def issue(rb, carry):
        vec = idx_v[pl.ds(rb * 16, 16)]
        for k in range(16):
            v = vec[k]
            pltpu.async_copy(
                tab_v.at[v], out_hbm.at[base + rb * 16 + k, pl.ds(col0, _DC)], sem
            )
        return carry

    lax.fori_loop(0, _ROWS_PER_G // 16, issue, 0)

    def drain(r, carry):
        pltpu.make_async_copy(
            tab_v.at[0], out_hbm.at[base + r, pl.ds(col0, _DC)], sem
        ).wait()
        return carry

    lax.fori_loop(0, _ROWS_PER_G, drain, 0)


def kernel(prefix, embedding_weight):
    idx = prefix.astype(jnp.int32).T.reshape(_NG, _ROWS_PER_G)
    out = _sc_gather(idx, embedding_weight)
    out = out.reshape(NUM_VIRTUAL_TOKENS, BATCH, EMBED_DIM)
    return out.transpose(1, 0, 2)


# speedup vs baseline: 5.8034x; 1.0537x over previous
import functools

import jax
import jax.numpy as jnp
from jax import lax
from jax.experimental import pallas as pl
from jax.experimental.pallas import tpu as pltpu
from jax.experimental.pallas import tpu_sc as plsc

NUM_VIRTUAL_TOKENS = 20
TOKEN_DIM = 768
NUM_LAYERS = 12
EMBED_DIM = 2 * NUM_LAYERS * TOKEN_DIM
BATCH = 128

_INFO = plsc.get_sparse_core_info()
_NC = _INFO.num_cores
_NS = _INFO.num_subcores
_NW = _NC * _NS

_B = BATCH * NUM_VIRTUAL_TOKENS
_NG = 4
_NCH = _NW // _NG
_ROWS_PER_G = _B // _NG
_DC = EMBED_DIM // _NCH

_SPLIT = 4
_FROM_SHARED = 0


@functools.partial(
    pl.kernel,
    mesh=plsc.VectorSubcoreMesh(core_axis_name="c", subcore_axis_name="s"),
    out_type=jax.ShapeDtypeStruct((_B, EMBED_DIM), jnp.float32),
    scratch_types=[
        pltpu.VMEM((_ROWS_PER_G,), jnp.int32),
        pltpu.VMEM((NUM_VIRTUAL_TOKENS, _DC), jnp.float32),
        pltpu.VMEM_SHARED((NUM_VIRTUAL_TOKENS, EMBED_DIM), jnp.float32),
        pltpu.SemaphoreType.DMA,
    ],
)
def _sc_gather(idx_hbm, table_hbm, out_hbm, idx_v, tab_v, tab_sh, sem):
    wid = lax.axis_index("c") * _NS + lax.axis_index("s")
    g = wid // _NCH
    c = wid % _NCH
    col0 = c * _DC
    base = g * _ROWS_PER_G
    g_local = lax.axis_index("s") // _NCH

    pltpu.sync_copy(idx_hbm.at[g], idx_v)
    pltpu.sync_copy(table_hbm.at[:, pl.ds(col0, _DC)], tab_v)

    @pl.when(g_local == 0)
    def _():
        pltpu.sync_copy(tab_v, tab_sh.at[:, pl.ds(col0, _DC)])

    plsc.subcore_barrier()

    def issue(rb, carry):
        vec = idx_v[pl.ds(rb * 16, 16)]
        for k in range(16):
            v = vec[k]
            dst = out_hbm.at[base + rb * 16 + k, pl.ds(col0, _DC)]
            if k % _SPLIT < _SPLIT - _FROM_SHARED:
                pltpu.async_copy(tab_v.at[v], dst, sem)
            else:
                pltpu.async_copy(tab_sh.at[v, pl.ds(col0, _DC)], dst, sem)
        return carry

    lax.fori_loop(0, _ROWS_PER_G // 16, issue, 0)

    def drain(r, carry):
        pltpu.make_async_copy(
            tab_v.at[0], out_hbm.at[base + r, pl.ds(col0, _DC)], sem
        ).wait()
        return carry

    lax.fori_loop(0, _ROWS_PER_G, drain, 0)


def kernel(prefix, embedding_weight):
    idx = prefix.astype(jnp.int32).T.reshape(_NG, _ROWS_PER_G)
    out = _sc_gather(idx, embedding_weight)
    out = out.reshape(NUM_VIRTUAL_TOKENS, BATCH, EMBED_DIM)
    return out.transpose(1, 0, 2)


# speedup vs baseline: 5.9462x; 1.0246x over previous
import functools

import jax
import jax.numpy as jnp
from jax import lax
from jax.experimental import pallas as pl
from jax.experimental.pallas import tpu as pltpu
from jax.experimental.pallas import tpu_sc as plsc

NUM_VIRTUAL_TOKENS = 20
TOKEN_DIM = 768
NUM_LAYERS = 12
EMBED_DIM = 2 * NUM_LAYERS * TOKEN_DIM
BATCH = 128

_INFO = plsc.get_sparse_core_info()
_NC = _INFO.num_cores
_NS = _INFO.num_subcores
_NW = _NC * _NS

_B = BATCH * NUM_VIRTUAL_TOKENS
_NG = 4
_NCH = _NW // _NG
_ROWS_PER_G = _B // _NG
_DC = EMBED_DIM // _NCH

@functools.partial(
    pl.kernel,
    mesh=plsc.VectorSubcoreMesh(core_axis_name="c", subcore_axis_name="s"),
    out_type=jax.ShapeDtypeStruct((_B, EMBED_DIM), jnp.float32),
    scratch_types=[
        pltpu.VMEM((_ROWS_PER_G,), jnp.int32),
        pltpu.VMEM((NUM_VIRTUAL_TOKENS, _DC), jnp.float32),
        pltpu.SemaphoreType.DMA,
    ],
)
def _sc_gather(idx_hbm, table_hbm, out_hbm, idx_v, tab_v, sem):
    wid = lax.axis_index("c") * _NS + lax.axis_index("s")
    g = wid // _NCH
    c = wid % _NCH
    col0 = c * _DC
    base = g * _ROWS_PER_G

    pltpu.sync_copy(idx_hbm.at[g], idx_v)
    pltpu.sync_copy(table_hbm.at[:, pl.ds(col0, _DC)], tab_v)

    def issue(rb, carry):
        vec = idx_v[pl.ds(rb * 16, 16)]
        for k in range(16):
            v = vec[k]
            dst = out_hbm.at[base + rb * 16 + k, pl.ds(col0, _DC)]
            pltpu.async_copy(tab_v.at[v], dst, sem)
        return carry

    lax.fori_loop(0, _ROWS_PER_G // 16, issue, 0)

    def drain(r, carry):
        pltpu.make_async_copy(
            tab_v.at[0], out_hbm.at[base + r, pl.ds(col0, _DC)], sem
        ).wait()
        return carry

    lax.fori_loop(0, _ROWS_PER_G, drain, 0)


def kernel(prefix, embedding_weight):
    idx = prefix.astype(jnp.int32).T.reshape(_NG, _ROWS_PER_G)
    out = _sc_gather(idx, embedding_weight)
    out = out.reshape(NUM_VIRTUAL_TOKENS, BATCH, EMBED_DIM)
    return out.transpose(1, 0, 2)
